# Initial kernel scaffold; baseline (speedup 1.0000x reference)
import jax
import jax.numpy as jnp
from jax.experimental import pallas as pl


def kernel(question, response, mask, user, q_neighbors, s_neighbors, q_neighbors_u, u_neighbors, emb_q, emb_s, emb_u, emb_q2, emb_r, W_in, W_h, w_pred):
    raise NotImplementedError("write your pallas kernel here")


# speedup vs baseline: 6.7347x; 6.7347x over previous
import functools
import jax
import jax.numpy as jnp
from jax import lax
from jax.experimental import pallas as pl
from jax.experimental.pallas import tpu as pltpu

_B, _L, _EMB, _K = 128, 32, 256, 10
_NEG = -3e38


def _xh_kernel(xin1_ref, xin2_ref, resp_ref, W1_ref, W2_ref, R_ref, Wh_ref,
               w1_ref, hw_ref, X_ref, hist_ref):
    R0 = R_ref[0:1, :]
    R1 = R_ref[1:2, :]
    X_ref[...] = (
        jnp.dot(xin1_ref[...], W1_ref[...], preferred_element_type=jnp.float32)
        + jnp.dot(xin2_ref[...], W2_ref[...], preferred_element_type=jnp.float32)
        + R0 + resp_ref[...] * (R1 - R0)
    )
    Wh = Wh_ref[...]

    def step(t, h):
        Xt = X_ref[pl.ds(t * _B, _B), :]
        h = jnp.tanh(Xt + jnp.dot(h, Wh, preferred_element_type=jnp.float32))
        hist_ref[pl.ds(t * _B, _B), :] = h
        return h

    lax.fori_loop(0, _L - 1, step, jnp.zeros((_B, _EMB), jnp.float32))
    hw_ref[...] = jnp.sum(hist_ref[...] * w1_ref[...], axis=1, keepdims=True)


def _run_xh(xin1, xin2, resp_tm, W1, W2, R, Wh, w1):
    return pl.pallas_call(
        _xh_kernel,
        out_shape=jax.ShapeDtypeStruct(((_L - 1) * _B, 1), jnp.float32),
        scratch_shapes=[pltpu.VMEM((_L * _B, _EMB), jnp.float32),
                        pltpu.VMEM(((_L - 1) * _B, _EMB), jnp.float32)],
    )(xin1, xin2, resp_tm, W1, W2, R, Wh, w1)


_BB = 8


def _att_kernel(eq_ref, eq2_ref, hwb_ref, w2_ref, y_ref):
    t_idx = lax.broadcasted_iota(jnp.int32, (_L - 1, _L), 0)
    tau_idx = lax.broadcasted_iota(jnp.int32, (_L - 1, _L), 1)
    valid = tau_idx <= t_idx
    w2 = w2_ref[...]
    for i in range(_BB):
        eqb = eq_ref[:, i, :]
        eq2b = eq2_ref[:, i, :]
        S = lax.dot_general(eqb, eqb, (((1,), (1,)), ((), ())),
                            preferred_element_type=jnp.float32)
        ScT = S[1:_L, :]
        cnt = jnp.zeros((_L - 1, _L), jnp.int32)
        for j in range(_L):
            sj = ScT[:, j:j + 1]
            vj = valid[:, j:j + 1]
            beats = (sj > ScT) | ((sj == ScT) & (j < tau_idx))
            cnt = cnt + jnp.where(vj & beats, 1, 0)
        keep = valid & (cnt < _K)
        Scm = jnp.where(keep, ScT, _NEG)
        mx = jnp.max(Scm, axis=1, keepdims=True)
        ex = jnp.where(keep, jnp.exp(ScT - mx), 0.0)
        att = ex / jnp.sum(ex, axis=1, keepdims=True)

        hwrow = hwb_ref[i:i + 1, :]
        hwpad = jnp.concatenate([hwrow, jnp.zeros((1, 1), jnp.float32)],
                                axis=1)
        recap = lax.dot_general(hwpad, att, (((1,), (1,)), ((), ())),
                                preferred_element_type=jnp.float32)
        su = eqb[1:_L, :] + eq2b[1:_L, :]
        c = lax.dot_general(w2, su, (((1,), (1,)), ((), ())),
                            preferred_element_type=jnp.float32)
        y = jax.nn.sigmoid(hwrow + recap + c)
        y_ref[i:i + 1, 0:1] = jnp.full((1, 1), 0.5, jnp.float32)
        y_ref[i:i + 1, 1:_L] = y


def _run_att(eq_tm, eq2_tm, hw_b, w2):
    return pl.pallas_call(
        _att_kernel,
        grid=(_B // _BB,),
        in_specs=[
            pl.BlockSpec((_L, _BB, _EMB), lambda g: (0, g, 0)),
            pl.BlockSpec((_L, _BB, _EMB), lambda g: (0, g, 0)),
            pl.BlockSpec((_BB, _L - 1), lambda g: (g, 0)),
            pl.BlockSpec((1, _EMB), lambda g: (0, 0)),
        ],
        out_specs=pl.BlockSpec((_BB, _L), lambda g: (g, 0)),
        out_shape=jax.ShapeDtypeStruct((_B, _L), jnp.float32),
    )(eq_tm, eq2_tm, hw_b, w2)


def kernel(question, response, mask, user, q_neighbors, s_neighbors,
           q_neighbors_u, u_neighbors,
           emb_q, emb_s, emb_u, emb_q2, emb_r, W_in, W_h, w_pred):
    question = question.astype(jnp.int32)
    response = response.astype(jnp.int32)
    mask = mask.astype(jnp.int32)
    user = user.astype(jnp.int32)

    T1 = jnp.tanh(emb_s + jnp.mean(emb_q[s_neighbors], axis=1))
    T2 = jnp.tanh(emb_u + jnp.mean(emb_q2[u_neighbors], axis=1))

    q_tm = question.T.reshape(-1)
    u_tm = user.T.reshape(-1)
    m_tm = mask.T.reshape(-1)
    eq = emb_q[q_tm]
    eq2 = emb_q2[q_tm]
    e_u = emb_u[u_tm]
    agg1 = jnp.mean(T1[q_neighbors[q_tm]], axis=1)
    aggu = jnp.mean(T2[q_neighbors_u[q_tm]], axis=1)
    mcol = (m_tm == 1)[:, None]
    xin1 = jnp.where(mcol, jnp.tanh(eq + agg1), eq) + e_u
    xin2 = jnp.tanh(eq2 + aggu)

    W1, W2, W3 = W_in[:_EMB], W_in[_EMB:2 * _EMB], W_in[2 * _EMB:]
    R = emb_r @ W3
    w1 = w_pred[:_EMB].reshape(1, _EMB)
    w2 = w_pred[_EMB:].reshape(1, _EMB)
    respf = response.T.reshape(-1, 1).astype(jnp.float32)
    hw_flat = _run_xh(xin1, xin2, respf, W1, W2, R, W_h, w1)
    hw_b = hw_flat.reshape(_L - 1, _B).T
    eq_tm = eq.reshape(_L, _B, _EMB)
    eq2_tm = eq2.reshape(_L, _B, _EMB)
    y = _run_att(eq_tm, eq2_tm, hw_b, w2)
    return y
